# baseline (speedup 1.0000x reference)
import functools

import jax
import jax.numpy as jnp
from jax import lax
from jax.experimental import pallas as pl
from jax.experimental.pallas import tpu as pltpu
from jax.experimental.pallas import tpu_sc as plsc

VOCAB = 1000
VPAD = 1024
BATCH = 16384

_info = plsc.get_sparse_core_info()
NC, NS = _info.num_cores, _info.num_subcores
NW = NC * NS
B_PER_W = BATCH // NW
R = 32
CH = B_PER_W // R


def _gather_kernel(table_pad, idx2):
    mesh = plsc.VectorSubcoreMesh(core_axis_name="c", subcore_axis_name="s")

    @functools.partial(
        pl.kernel,
        mesh=mesh,
        out_type=jax.ShapeDtypeStruct((BATCH, VPAD), jnp.float32),
        scratch_types=[
            pltpu.VMEM((CH, R), jnp.int32),
            pltpu.VMEM((R, VPAD), jnp.float32),
            pltpu.SemaphoreType.DMA,
            pltpu.SemaphoreType.DMA,
        ],
    )
    def k(table_hbm, idx_hbm, out_hbm, idx_v, buf, gsem, wsem):
        sid = lax.axis_index("s")
        wid = sid * NC + lax.axis_index("c")
        base = wid * B_PER_W
        pltpu.sync_copy(idx_hbm.at[pl.ds(wid * CH, CH)], idx_v)

        def chunk(c, carry):
            pltpu.async_copy(table_hbm.at[idx_v.at[c]], buf, gsem).wait()
            pltpu.async_copy(
                buf, out_hbm.at[pl.ds(base + c * R, R)], wsem
            ).wait()
            return carry

        lax.fori_loop(0, CH, chunk, 0)

    return k(table_pad, idx2)


def kernel(idx, token_embedding_table):
    table_pad = jnp.pad(token_embedding_table, ((0, 0), (0, VPAD - VOCAB)))
    idx2 = idx.reshape(NW * CH, R)
    out_pad = _gather_kernel(table_pad, idx2)
    return out_pad[:, :VOCAB]
